# baseline (speedup 1.0000x reference)
import functools

import numpy as np

import jax
import jax.numpy as jnp
from jax import lax
from jax.experimental import pallas as pl
from jax.experimental.pallas import tpu as pltpu
from jax.experimental.pallas import tpu_sc as plsc

B = 16384
F = 16
PACK = 8
NC = 2
NS = 16
NW = NC * NS
BPW = B // NW
CHUNK = 128
NCHUNK = BPW // CHUNK
L = 16

_MESH = plsc.VectorSubcoreMesh(core_axis_name="c", subcore_axis_name="s")


def _extract_chunk(gbuf, idx_v, out_v, c):
    def body(jg, carry):
        iota = lax.iota(jnp.int32, L)
        row0 = c * CHUNK + jg * L
        idv = idx_v[pl.ds(row0, L)]
        off = (idv & (PACK - 1)) << 4
        jrow = jg * L + iota
        orow = row0 + iota
        prow = orow >> 3
        pcol = (orow & (PACK - 1)) << 4
        for k in range(F):
            vals = plsc.load_gather(gbuf, [jrow, off + k])
            plsc.store_scatter(out_v, [prow, pcol + k], vals)
        return carry

    lax.fori_loop(0, CHUNK // L, body, 0)


@functools.partial(
    pl.kernel,
    out_type=[
        jax.ShapeDtypeStruct((B // PACK, 128), jnp.float32),
        jax.ShapeDtypeStruct((B // PACK, 128), jnp.float32),
    ],
    mesh=_MESH,
    compiler_params=pltpu.CompilerParams(needs_layout_passes=False),
    scratch_types=[
        pltpu.VMEM((BPW,), jnp.int32),
        pltpu.VMEM((BPW,), jnp.int32),
        pltpu.VMEM((BPW,), jnp.int32),
        pltpu.VMEM((BPW,), jnp.int32),
        pltpu.VMEM((2, CHUNK, 128), jnp.float32),
        pltpu.VMEM((2, CHUNK, 128), jnp.float32),
        pltpu.VMEM((BPW // PACK, 128), jnp.float32),
        pltpu.VMEM((BPW // PACK, 128), jnp.float32),
        pltpu.SemaphoreType.DMA,
    ],
)
def _sc_gather(user_h, movie_h, utp_h, mtp_h, uo_h, mo_h,
               uidx_v, midx_v, updx_v, mpdx_v, gu_v, gm_v, ou_v, om_v, sem):
    wid = lax.axis_index("s") * NC + lax.axis_index("c")
    base = wid * BPW
    pltpu.sync_copy(user_h.at[pl.ds(base, BPW)], uidx_v)
    pltpu.sync_copy(movie_h.at[pl.ds(base, BPW)], midx_v)
    for i in range(BPW // L):
        sl = pl.ds(i * L, L)
        updx_v[sl] = uidx_v[sl] >> 3
        mpdx_v[sl] = midx_v[sl] >> 3

    def fire(c):
        sl = pl.ds(c * CHUNK, CHUNK)
        return (
            pltpu.async_copy(utp_h.at[updx_v.at[sl]], gu_v.at[c % 2], sem),
            pltpu.async_copy(mtp_h.at[mpdx_v.at[sl]], gm_v.at[c % 2], sem),
        )

    pending = fire(0)
    for c in range(NCHUNK):
        for cp in pending:
            cp.wait()
        if c + 1 < NCHUNK:
            pending = fire(c + 1)
        _extract_chunk(gu_v.at[c % 2], uidx_v, ou_v, c)
        _extract_chunk(gm_v.at[c % 2], midx_v, om_v, c)
    obase = wid * (BPW // PACK)
    pltpu.sync_copy(ou_v, uo_h.at[pl.ds(obase, BPW // PACK)])
    pltpu.sync_copy(om_v, mo_h.at[pl.ds(obase, BPW // PACK)])


def _mlp_body(u_ref, m_ref, w1u_ref, w1m_ref, b1_ref, w2_ref, b2_ref, o_ref):
    h = jnp.dot(u_ref[...], w1u_ref[...], preferred_element_type=jnp.float32)
    h = h + jnp.dot(m_ref[...], w1m_ref[...], preferred_element_type=jnp.float32)
    h = jnp.maximum(h + b1_ref[...], 0.0)
    o = jnp.dot(h, w2_ref[...], preferred_element_type=jnp.float32) + b2_ref[...]
    o_ref[...] = 5.5 / (1.0 + jnp.exp(-o))


def _mlp(u_emb, m_emb, w1u, w1m, b1, w2, b2):
    return pl.pallas_call(
        _mlp_body,
        out_shape=jax.ShapeDtypeStruct((B, 1), jnp.float32),
    )(u_emb, m_emb, w1u, w1m, b1, w2, b2)


def kernel(user, movie, u_table, m_table, W1, b1, W2, b2):
    user = user.astype(jnp.int32)
    movie = movie.astype(jnp.int32)
    utp = u_table.reshape(-1, 128)
    mtp = m_table.reshape(-1, 128)
    uo, mo = _sc_gather(user, movie, utp, mtp)
    u_emb = uo.reshape(B, F)
    m_emb = mo.reshape(B, F)
    return _mlp(u_emb, m_emb, W1[:F], W1[F:], b1.reshape(1, -1), W2,
                b2.reshape(1, 1))
